# Initial kernel scaffold; baseline (speedup 1.0000x reference)
import jax
import jax.numpy as jnp
from jax.experimental import pallas as pl


def kernel(x, edge_index, edge_attr, batch, params):
    raise NotImplementedError("write your pallas kernel here")


# speedup vs baseline: 1.0012x; 1.0012x over previous
import jax
import jax.numpy as jnp
from jax.experimental import pallas as pl


def _leaky(v):
    return jnp.where(v >= 0, v, 0.01 * v)


def _mlp_body(z_ref, w1, b1, w2, b2, w3, b3, o_ref):
    z = z_ref[...]
    z = _leaky(z @ w1[...] + b1[...])
    z = _leaky(z @ w2[...] + b2[...])
    o_ref[...] = z @ w3[...] + b3[...]


def kernel(x, edge_index, edge_attr, batch, params):
    src = edge_index[0]
    dst = edge_index[1]
    n = x.shape[0]
    NG = 9
    cnt = jax.ops.segment_sum(jnp.ones((edge_index.shape[1],), x.dtype), dst, num_segments=n)
    deg = jnp.clip(cnt, 1.0, None)
    avg_lin = jnp.mean(deg)
    avg_log = jnp.mean(jnp.log(deg + 1.0))
    has_edge = (cnt > 0)[:, None]
    d = deg[:, None]
    logd = jnp.log(d + 1.0)
    for p in params['convs']:
        e = edge_attr @ p['eW'] + p['eb']
        h = jnp.concatenate([x[dst], x[src], e], axis=-1)
        m = h @ p['pW'] + p['pb']
        mean = jax.ops.segment_sum(m, dst, num_segments=n) / d
        mean_sq = jax.ops.segment_sum(m * m, dst, num_segments=n) / d
        mn = jnp.where(has_edge, jax.ops.segment_min(m, dst, num_segments=n), 0.0)
        mx = jnp.where(has_edge, jax.ops.segment_max(m, dst, num_segments=n), 0.0)
        std = jnp.sqrt(jax.nn.relu(mean_sq - mean * mean) + 1e-5)
        agg = jnp.concatenate([mean, mn, mx, std], axis=-1)
        scaled = jnp.concatenate([agg, agg * (logd / avg_log), agg * (avg_log / logd), agg * (d / avg_lin)], axis=-1)
        out = jnp.concatenate([x, scaled], axis=-1) @ p['oW'] + p['ob']
        out = out @ p['lW'] + p['lb']
        mu = jnp.mean(out, axis=0)
        var = jnp.var(out, axis=0)
        out = (out - mu) / jnp.sqrt(var + 1e-5) * p['bn_g'] + p['bn_b']
        x = _leaky(out)
    gcnt = jax.ops.segment_sum(jnp.ones((n,), x.dtype), batch, num_segments=NG)
    x1 = jax.ops.segment_sum(x, batch, num_segments=NG) / jnp.clip(gcnt, 1.0, None)[:, None]
    x2 = jnp.where((gcnt > 0)[:, None], jax.ops.segment_max(x, batch, num_segments=NG), 0.0)
    z = jnp.concatenate([x1, x2], axis=-1)
    out = pl.pallas_call(
        _mlp_body,
        out_shape=jax.ShapeDtypeStruct((NG, 1), jnp.float32),
    )(z, params['l1W'], params['l1b'], params['l2W'], params['l2b'],
      params['l3W'], params['l3b'])
    return out


# speedup vs baseline: 2.5957x; 2.5925x over previous
import jax
import jax.numpy as jnp
from jax import lax
from jax.experimental import pallas as pl
from jax.experimental.pallas import tpu as pltpu
from jax.experimental.pallas import tpu_sc as plsc

NN = 10000
EE = 160000
NGRP = 9
NW = 32
NV = 313
CHUNK = 128
EPAD = 160256
EB = 2048
NB = 1000
GRID = NN // NB
NTOT = NW * NV
BIGF = 3.0e38

EPAD = 160 * 1024


def _sc_stats_call(xd, xs, em, sdst, ssrc, bnds, Fh, with_cnt):
    KF = Fh // 16
    mesh = plsc.VectorSubcoreMesh(core_axis_name="c", subcore_axis_name="s")
    out_type = [jax.ShapeDtypeStruct((NW, NV * Fh), jnp.float32)] * 4
    if with_cnt:
        out_type.append(jax.ShapeDtypeStruct((NW, NV * 16), jnp.float32))
    scratch = [pltpu.VMEM((NV * Fh,), jnp.float32) for _ in range(4)]
    if with_cnt:
        scratch.append(pltpu.VMEM((NV * 16,), jnp.float32))
    scratch += [
        pltpu.VMEM((CHUNK,), jnp.int32),
        pltpu.VMEM((CHUNK,), jnp.int32),
        pltpu.VMEM((CHUNK, Fh), jnp.float32),
        pltpu.VMEM((CHUNK, Fh), jnp.float32),
        pltpu.VMEM((CHUNK, Fh), jnp.float32),
        pltpu.VMEM((48,), jnp.int32),
        pltpu.SemaphoreType.DMA,
        pltpu.SemaphoreType.DMA,
    ]
    ns = 5 if with_cnt else 4

    def body(xd_h, xs_h, em_h, dst_h, src_h, bnds_h, *rest):
        outs = rest[:ns]
        accs = rest[ns:2 * ns]
        (v_dst, v_src, v_xd, v_xs, v_em, v_bnds, sem1,
         sem2) = rest[2 * ns:]
        cid = lax.axis_index("c")
        sid = lax.axis_index("s")
        wid = sid * 2 + cid
        v0 = wid * NV
        pltpu.sync_copy(bnds_h, v_bnds)
        bidx = lax.iota(jnp.int32, 16) + wid
        bwin = plsc.load_gather(v_bnds, [bidx])
        b0 = bwin[0]
        b1 = bwin[1]
        b0a = (b0 // 8) * 8
        nch = (b1 - b0a + CHUNK - 1) // CHUNK

        zv = jnp.zeros((16,), jnp.float32)
        lov = jnp.full((16,), -BIGF, jnp.float32)
        hiv = jnp.full((16,), BIGF, jnp.float32)
        ones = jnp.ones((16,), jnp.float32)

        def init_body(i, carry):
            off = i * 16
            accs[0][pl.ds(off, 16)] = zv
            accs[1][pl.ds(off, 16)] = zv
            accs[2][pl.ds(off, 16)] = hiv
            accs[3][pl.ds(off, 16)] = lov
            return carry
        lax.fori_loop(0, NV * KF, init_body, 0)
        if with_cnt:
            def initc(i, carry):
                accs[4][pl.ds(i * 16, 16)] = zv
                return carry
            lax.fori_loop(0, NV, initc, 0)

        def chunk_body(ci, carry):
            base = b0a + ci * CHUNK
            pltpu.sync_copy(dst_h.at[pl.ds(base, CHUNK)], v_dst)
            pltpu.sync_copy(src_h.at[pl.ds(base, CHUNK)], v_src)
            cp1 = pltpu.async_copy(xd_h.at[v_dst], v_xd, sem1)
            cp2 = pltpu.async_copy(xs_h.at[v_src], v_xs, sem2)
            pltpu.sync_copy(em_h.at[pl.ds(base, CHUNK)], v_em)
            cp1.wait()
            cp2.wait()

            def group_body(q, carry2):
                e0 = q * 16
                dvec = v_dst[pl.ds(e0, 16)]
                for j in range(16):
                    g = base + e0 + j

                    @pl.when(jnp.logical_and(g >= b0, g < b1))
                    def _():
                        dl = dvec[j] - v0
                        off = dl * Fh
                        e = e0 + j
                        for k in range(KF):
                            col = k * 16
                            mk = (v_xd[e, pl.ds(col, 16)]
                                  + v_xs[e, pl.ds(col, 16)]
                                  + v_em[e, pl.ds(col, 16)])
                            so = off + col
                            accs[0][pl.ds(so, 16)] = accs[0][pl.ds(so, 16)] + mk
                            accs[1][pl.ds(so, 16)] = (accs[1][pl.ds(so, 16)]
                                                      + mk * mk)
                            accs[2][pl.ds(so, 16)] = jnp.minimum(
                                accs[2][pl.ds(so, 16)], mk)
                            accs[3][pl.ds(so, 16)] = jnp.maximum(
                                accs[3][pl.ds(so, 16)], mk)
                        if with_cnt:
                            co = dl * 16
                            accs[4][pl.ds(co, 16)] = (accs[4][pl.ds(co, 16)]
                                                      + ones)
                return carry2
            lax.fori_loop(0, CHUNK // 16, group_body, 0)
            return carry
        lax.fori_loop(0, nch, chunk_body, 0)
        for j in range(ns):
            pltpu.sync_copy(accs[j], outs[j].at[wid])

    fn = pl.kernel(body, out_type=tuple(out_type), mesh=mesh,
                   scratch_types=tuple(scratch),
                   compiler_params=pltpu.CompilerParams(
                       use_tc_tiling_on_sc=False,
                       needs_layout_passes=False))
    return fn(xd, xs, em, sdst, ssrc, bnds)


def _full(shape):
    return pl.BlockSpec(shape, lambda i: tuple(0 for _ in shape))


def _rows(nb, f):
    return pl.BlockSpec((nb, f), lambda i: (i, 0))


def _prep_body(x_ref, pd_ref, ps_ref, xd_ref, xs_ref):
    xb = x_ref[...]
    xd_ref[...] = jnp.dot(xb, pd_ref[...], preferred_element_type=jnp.float32, precision=lax.Precision.HIGHEST)
    xs_ref[...] = jnp.dot(xb, ps_ref[...], preferred_element_type=jnp.float32, precision=lax.Precision.HIGHEST)


def _prep_call(x, pd, ps):
    f = pd.shape[1]
    return pl.pallas_call(
        _prep_body,
        grid=(GRID,),
        in_specs=[_rows(NB, x.shape[1]), _full(pd.shape), _full(ps.shape)],
        out_specs=[_rows(NB, f), _rows(NB, f)],
        out_shape=[jax.ShapeDtypeStruct((NN, f), jnp.float32)] * 2,
    )(x, pd, ps)


def _em16_body(ea_ref, M_ref, c_ref, o_ref):
    o_ref[...] = jnp.dot(ea_ref[...], M_ref[...],
                         preferred_element_type=jnp.float32, precision=lax.Precision.HIGHEST) + c_ref[...]


def _em16_call(sea_p, M, c):
    return pl.pallas_call(
        _em16_body,
        grid=(EPAD // EB,),
        in_specs=[_rows(EB, 4), _full((4, 16)), _full((1, 16))],
        out_specs=_rows(EB, 16),
        out_shape=jax.ShapeDtypeStruct((EPAD, 16), jnp.float32),
    )(sea_p, M, c.reshape(1, 16))


def _em128_body(ea_ref, M_ref, c_ref, lo_ref, hi_ref):
    em = jnp.dot(ea_ref[...], M_ref[...],
                 preferred_element_type=jnp.float32, precision=lax.Precision.HIGHEST) + c_ref[...]
    lo_ref[...] = em[:, 0:64]
    hi_ref[...] = em[:, 64:128]


def _em128_call(sea_p, M, c):
    return pl.pallas_call(
        _em128_body,
        grid=(EPAD // EB,),
        in_specs=[_rows(EB, 4), _full((4, 128)), _full((1, 128))],
        out_specs=[_rows(EB, 64), _rows(EB, 64)],
        out_shape=[jax.ShapeDtypeStruct((EPAD, 64), jnp.float32)] * 2,
    )(sea_p, M, c.reshape(1, 128))


def _deg_body(cnt_ref, dcols_ref, sums_ref):
    i = pl.program_id(0)
    c = cnt_ref[...]
    deg = jnp.maximum(c, 1.0)
    logd = jnp.log(deg + 1.0)
    has = (c > 0).astype(jnp.float32)
    dcols_ref[...] = jnp.concatenate(
        [deg, logd, has, jnp.zeros_like(c)], axis=1)
    blk = jnp.concatenate(
        [jnp.sum(deg).reshape(1, 1), jnp.sum(logd).reshape(1, 1),
         jnp.zeros((1, 6), jnp.float32)], axis=1)

    @pl.when(i == 0)
    def _():
        sums_ref[...] = jnp.zeros_like(sums_ref)
    sums_ref[...] += blk


def _deg_call(cnt):
    return pl.pallas_call(
        _deg_body,
        grid=(GRID,),
        in_specs=[_rows(NB, 1)],
        out_specs=[_rows(NB, 4), _full((1, 8))],
        out_shape=[jax.ShapeDtypeStruct((NN, 4), jnp.float32),
                   jax.ShapeDtypeStruct((1, 8), jnp.float32)],
    )(cnt)


def _make_conv_body(f, nseg):
    def body(*refs):
        srefs = refs[:4 * nseg]
        (x_ref, dc_ref, sums_ref, oWx_ref, oWcat_ref, ob_ref, lW_ref,
         lb_ref, o_ref, bn_ref) = refs[4 * nseg:]
        i = pl.program_id(0)
        dc = dc_ref[...]
        deg = dc[:, 0:1]
        logd = dc[:, 1:2]
        has = dc[:, 2:3]
        avg_lin = sums_ref[0, 0] / NN
        avg_log = sums_ref[0, 1] / NN

        def cat(j):
            v = jnp.concatenate([srefs[j * nseg + s][...]
                                 for s in range(nseg)], axis=1) if nseg > 1 \
                else srefs[j][...]
            return v[:, :f]
        ssum = cat(0)
        ssq = cat(1)
        mean = ssum / deg
        meansq = ssq / deg
        std = jnp.sqrt(jnp.maximum(meansq - mean * mean, 0.0) + 1e-5)
        mn = jnp.where(has > 0, cat(2), 0.0)
        mx = jnp.where(has > 0, cat(3), 0.0)
        agg = jnp.concatenate([mean, mn, mx, std], axis=1)
        Rm = jnp.dot(agg, oWcat_ref[...], preferred_element_type=jnp.float32, precision=lax.Precision.HIGHEST)
        s1 = logd / avg_log
        s2 = avg_log / logd
        s3 = deg / avg_lin
        out = (jnp.dot(x_ref[...], oWx_ref[...],
                       preferred_element_type=jnp.float32, precision=lax.Precision.HIGHEST)
               + Rm[:, 0:128] + s1 * Rm[:, 128:256] + s2 * Rm[:, 256:384]
               + s3 * Rm[:, 384:512] + ob_ref[...])
        out = jnp.dot(out, lW_ref[...],
                      preferred_element_type=jnp.float32, precision=lax.Precision.HIGHEST) + lb_ref[...]
        o_ref[...] = out

        @pl.when(i == 0)
        def _():
            bn_ref[...] = jnp.zeros_like(bn_ref)
        bn_ref[...] += jnp.concatenate(
            [jnp.sum(out, 0, keepdims=True),
             jnp.sum(out * out, 0, keepdims=True)], axis=0)
    return body


def _conv_call(stat_arrays, xc, dcols, sums, oWx, oWcat, ob, lW, lb, f, nseg):
    fp = stat_arrays[0].shape[1]
    in_specs = ([_rows(NB, fp)] * (4 * nseg)
                + [_rows(NB, xc.shape[1]), _rows(NB, 4), _full((1, 8)),
                   _full(oWx.shape), _full(oWcat.shape), _full((1, 128)),
                   _full((128, 128)), _full((1, 128))])
    return pl.pallas_call(
        _make_conv_body(f, nseg),
        grid=(GRID,),
        in_specs=in_specs,
        out_specs=[_rows(NB, 128), _full((2, 128))],
        out_shape=[jax.ShapeDtypeStruct((NN, 128), jnp.float32),
                   jax.ShapeDtypeStruct((2, 128), jnp.float32)],
    )(*stat_arrays, xc, dcols, sums, oWx, oWcat, ob.reshape(1, 128), lW,
      lb.reshape(1, 128))


def _fin_body(o_ref, bn_ref, g_ref, b_ref, pd_ref, ps_ref,
              xn_ref, xdlo_ref, xdhi_ref, xslo_ref, xshi_ref):
    mu = bn_ref[0:1, :] / NN
    var = bn_ref[1:2, :] / NN - mu * mu
    inv = lax.rsqrt(var + 1e-5)
    o = (o_ref[...] - mu) * inv * g_ref[...] + b_ref[...]
    xn = jnp.where(o >= 0, o, 0.01 * o)
    xn_ref[...] = xn
    xd = jnp.dot(xn, pd_ref[...], preferred_element_type=jnp.float32, precision=lax.Precision.HIGHEST)
    xs = jnp.dot(xn, ps_ref[...], preferred_element_type=jnp.float32, precision=lax.Precision.HIGHEST)
    xdlo_ref[...] = xd[:, 0:64]
    xdhi_ref[...] = xd[:, 64:128]
    xslo_ref[...] = xs[:, 0:64]
    xshi_ref[...] = xs[:, 64:128]


def _fin_call(out2, bn, g, b, pd, ps):
    return pl.pallas_call(
        _fin_body,
        grid=(GRID,),
        in_specs=[_rows(NB, 128), _full((2, 128)), _full((1, 128)),
                  _full((1, 128)), _full((128, 128)), _full((128, 128))],
        out_specs=[_rows(NB, 128)] + [_rows(NB, 64)] * 4,
        out_shape=[jax.ShapeDtypeStruct((NN, 128), jnp.float32)]
        + [jax.ShapeDtypeStruct((NN, 64), jnp.float32)] * 4,
    )(out2, bn, g.reshape(1, 128), b.reshape(1, 128), pd, ps)


def _finlast_body(o_ref, bn_ref, g_ref, b_ref, xn_ref):
    mu = bn_ref[0:1, :] / NN
    var = bn_ref[1:2, :] / NN - mu * mu
    inv = lax.rsqrt(var + 1e-5)
    o = (o_ref[...] - mu) * inv * g_ref[...] + b_ref[...]
    xn_ref[...] = jnp.where(o >= 0, o, 0.01 * o)


def _finlast_call(out2, bn, g, b):
    return pl.pallas_call(
        _finlast_body,
        grid=(GRID,),
        in_specs=[_rows(NB, 128), _full((2, 128)), _full((1, 128)),
                  _full((1, 128))],
        out_specs=_rows(NB, 128),
        out_shape=jax.ShapeDtypeStruct((NN, 128), jnp.float32),
    )(out2, bn, g.reshape(1, 128), b.reshape(1, 128))


def _pool_body(x_ref, b_ref, w1_ref, b1_ref, w2_ref, b2_ref, w3_ref, b3_ref,
               out_ref, s_sum, s_max, s_cnt):
    i = pl.program_id(0)

    @pl.when(i == 0)
    def _():
        s_sum[...] = jnp.zeros_like(s_sum)
        s_max[...] = jnp.full_like(s_max, -BIGF)
        s_cnt[...] = jnp.zeros_like(s_cnt)

    xb = x_ref[...]
    bb = b_ref[...]
    for g in range(NGRP):
        mask = bb == g
        s = jnp.sum(jnp.where(mask, xb, 0.0), axis=0, keepdims=True)
        mx = jnp.max(jnp.where(mask, xb, -BIGF), axis=0, keepdims=True)
        cg = jnp.sum(mask.astype(jnp.float32))
        s_sum[g:g + 1, :] += s
        s_max[g:g + 1, :] = jnp.maximum(s_max[g:g + 1, :], mx)
        s_cnt[g:g + 1, :] += jnp.full((1, 128), 1.0, jnp.float32) * cg

    @pl.when(i == GRID - 1)
    def _():
        cnt = s_cnt[...]
        x1 = s_sum[...] / jnp.maximum(cnt, 1.0)
        x2 = jnp.where(cnt > 0, s_max[...], 0.0)
        z = jnp.concatenate([x1, x2], axis=1)
        z = jnp.dot(z, w1_ref[...], preferred_element_type=jnp.float32, precision=lax.Precision.HIGHEST) + b1_ref[...]
        z = jnp.where(z >= 0, z, 0.01 * z)
        z = jnp.dot(z, w2_ref[...], preferred_element_type=jnp.float32, precision=lax.Precision.HIGHEST) + b2_ref[...]
        z = jnp.where(z >= 0, z, 0.01 * z)
        o = jnp.dot(z, w3_ref[...], preferred_element_type=jnp.float32, precision=lax.Precision.HIGHEST) + b3_ref[...]
        out_ref[...] = o[:NGRP, :]


def _pool_call(xc, batch2, l1W, l1b, l2W, l2b, l3W, l3b):
    return pl.pallas_call(
        _pool_body,
        grid=(GRID,),
        in_specs=[_rows(NB, 128), _rows(NB, 1), _full((256, 128)),
                  _full((1, 128)), _full((128, 64)), _full((1, 64)),
                  _full((64, 1)), _full((1, 1))],
        out_specs=_full((NGRP, 1)),
        out_shape=jax.ShapeDtypeStruct((NGRP, 1), jnp.float32),
        scratch_shapes=[pltpu.VMEM((16, 128), jnp.float32),
                        pltpu.VMEM((16, 128), jnp.float32),
                        pltpu.VMEM((16, 128), jnp.float32)],
    )(xc, batch2, l1W, l1b.reshape(1, 128), l2W, l2b.reshape(1, 64), l3W,
      l3b.reshape(1, 1))


def kernel(x, edge_index, edge_attr, batch, params):
    src = edge_index[0]
    dst = edge_index[1]
    perm = jnp.argsort(dst)
    sdst = dst[perm]
    ssrc = src[perm]
    sea = edge_attr[perm]
    pad = EPAD - EE
    sdst_p = jnp.concatenate([sdst, jnp.zeros((pad,), jnp.int32)])
    ssrc_p = jnp.concatenate([ssrc, jnp.zeros((pad,), jnp.int32)])
    sea_p = jnp.concatenate([sea, jnp.zeros((pad, 4), jnp.float32)])
    ranges = jnp.minimum(jnp.arange(NW + 1, dtype=jnp.int32) * NV, NN)
    bnds = jnp.searchsorted(sdst, ranges, side='left').astype(jnp.int32)
    bnds_p = jnp.concatenate([bnds, jnp.full((48 - NW - 1,), EE, jnp.int32)])
    convs = params['convs']

    p0 = convs[0]
    pW3_0 = p0['pW'][8:12]
    pd0 = jnp.pad(p0['pW'][0:4], ((0, 0), (0, 12)))
    ps0 = jnp.pad(p0['pW'][4:8], ((0, 0), (0, 12)))
    M0 = jnp.pad(p0['eW'] @ pW3_0, ((0, 0), (0, 12)))
    c0 = jnp.pad(p0['eb'] @ pW3_0 + p0['pb'], (0, 12))
    xd0, xs0 = _prep_call(x, pd0, ps0)
    em0 = _em16_call(sea_p, M0, c0)
    ssum, ssq, smn, smx, scnt = _sc_stats_call(
        xd0, xs0, em0, sdst_p, ssrc_p, bnds_p, 16, True)
    cnt = scnt.reshape(NTOT, 16)[:NN, 0:1]
    dcols, sums = _deg_call(cnt)
    stat_arrays = [ssum.reshape(NTOT, 16), ssq.reshape(NTOT, 16),
                   smn.reshape(NTOT, 16), smx.reshape(NTOT, 16)]
    nseg = 1

    xc = x
    for li in range(6):
        p = convs[li]
        f = 4 if li == 0 else 128
        oWx = p['oW'][:f]
        A = p['oW'][f:]
        oWcat = jnp.concatenate(
            [A[4 * f * k:4 * f * (k + 1)] for k in range(4)], axis=1)
        out2, bn = _conv_call(stat_arrays, xc, dcols, sums, oWx, oWcat,
                              p['ob'], p['lW'], p['lb'], f, nseg)
        if li < 5:
            pn = convs[li + 1]
            pd = pn['pW'][0:128]
            ps = pn['pW'][128:256]
            xc, xdlo, xdhi, xslo, xshi = _fin_call(
                out2, bn, p['bn_g'], p['bn_b'], pd, ps)
            Mn = pn['eW'] @ pn['pW'][256:384]
            cn = pn['eb'] @ pn['pW'][256:384] + pn['pb']
            em_lo, em_hi = _em128_call(sea_p, Mn, cn)
            r_lo = _sc_stats_call(xdlo, xslo, em_lo, sdst_p, ssrc_p, bnds_p,
                                  64, False)
            r_hi = _sc_stats_call(xdhi, xshi, em_hi, sdst_p, ssrc_p, bnds_p,
                                  64, False)
            stat_arrays = []
            for a, b in zip(r_lo, r_hi):
                stat_arrays.append(a.reshape(NTOT, 64))
                stat_arrays.append(b.reshape(NTOT, 64))
            nseg = 2
        else:
            xc = _finlast_call(out2, bn, p['bn_g'], p['bn_b'])

    batch2 = batch.reshape(NN, 1)
    return _pool_call(xc, batch2, params['l1W'], params['l1b'],
                      params['l2W'], params['l2b'], params['l3W'],
                      params['l3b'])
